# baseline (speedup 1.0000x reference)
import jax
import jax.numpy as jnp
from jax.experimental import pallas as pl
from jax.experimental.pallas import tpu as pltpu

_B, _D, _E, _K = 2048, 768, 8, 2
_BLK = 512


def _moe_kernel(x_ref, wrt_ref, rb_ref, we_ref, be_ref,
                out_ref, loss_ref, probs_acc, mask_acc):
    i = pl.program_id(0)
    nblocks = pl.num_programs(0)

    x = x_ref[...]

    logits = jnp.dot(x, wrt_ref[...],
                     preferred_element_type=jnp.float32) + rb_ref[...]
    m = jnp.max(logits, axis=-1, keepdims=True)
    ex = jnp.exp(logits - m)
    probs = ex / jnp.sum(ex, axis=-1, keepdims=True)

    eidx = jax.lax.broadcasted_iota(jnp.int32, probs.shape, 1)
    a1 = jnp.argmax(probs, axis=-1)[:, None]
    oh1 = (eidx == a1).astype(jnp.float32)
    w1 = jnp.max(probs, axis=-1, keepdims=True)
    probs2 = jnp.where(oh1 > 0, -jnp.inf, probs)
    a2 = jnp.argmax(probs2, axis=-1)[:, None]
    oh2 = (eidx == a2).astype(jnp.float32)
    w2 = jnp.max(probs2, axis=-1, keepdims=True)

    denom = w1 + w2
    combine = (w1 / denom) * oh1 + (w2 / denom) * oh2
    mask = oh1 + oh2

    @pl.when(i == 0)
    def _():
        probs_acc[...] = jnp.zeros_like(probs_acc)
        mask_acc[...] = jnp.zeros_like(mask_acc)

    probs_acc[...] += jnp.sum(probs, axis=0, keepdims=True)
    mask_acc[...] += jnp.sum(mask, axis=0, keepdims=True)

    @pl.when(i == nblocks - 1)
    def _():
        loss_ref[...] = jnp.sum(
            probs_acc[...] * mask_acc[...], keepdims=True) / (_B * _B)

    acc = jnp.dot(combine, be_ref[...],
                  preferred_element_type=jnp.float32)
    for e in range(_E):
        y = jax.lax.dot_general(
            x, we_ref[e],
            dimension_numbers=(((1,), (1,)), ((), ())),
            preferred_element_type=jnp.float32)
        acc = acc + combine[:, e][:, None] * y
    out_ref[...] = acc


def kernel(x, context_vector, Wr, br, We, be, context_weight):
    rb = (br + context_weight * context_vector).reshape(1, _E)
    wrt = Wr.T

    grid = (_B // _BLK,)
    out, loss = pl.pallas_call(
        _moe_kernel,
        grid=grid,
        in_specs=[
            pl.BlockSpec((_BLK, _D), lambda i: (i, 0)),
            pl.BlockSpec((_D, _E), lambda i: (0, 0)),
            pl.BlockSpec((1, _E), lambda i: (0, 0)),
            pl.BlockSpec((_E, _D, _D), lambda i: (0, 0, 0)),
            pl.BlockSpec((_E, _D), lambda i: (0, 0)),
        ],
        out_specs=[
            pl.BlockSpec((_BLK, _D), lambda i: (i, 0)),
            pl.BlockSpec((1, 1), lambda i: (0, 0)),
        ],
        out_shape=[
            jax.ShapeDtypeStruct((_B, _D), jnp.float32),
            jax.ShapeDtypeStruct((1, 1), jnp.float32),
        ],
        scratch_shapes=[
            pltpu.VMEM((1, _E), jnp.float32),
            pltpu.VMEM((1, _E), jnp.float32),
        ],
    )(x, wrt, rb, We, be)
    return out, loss[0, 0]


# speedup vs baseline: 1.0424x; 1.0424x over previous
import jax
import jax.numpy as jnp
from jax.experimental import pallas as pl
from jax.experimental.pallas import tpu as pltpu

_B, _D, _E, _K = 2048, 768, 8, 2
_BLK = 2048


def _moe_kernel(x_ref, wrt_ref, rb_ref, we_ref, be_ref,
                out_ref, loss_ref, probs_acc, mask_acc):
    i = pl.program_id(0)
    nblocks = pl.num_programs(0)

    x = x_ref[...]

    logits = jnp.dot(x, wrt_ref[...],
                     preferred_element_type=jnp.float32) + rb_ref[...]
    m = jnp.max(logits, axis=-1, keepdims=True)
    ex = jnp.exp(logits - m)
    probs = ex / jnp.sum(ex, axis=-1, keepdims=True)

    eidx = jax.lax.broadcasted_iota(jnp.int32, probs.shape, 1)
    a1 = jnp.argmax(probs, axis=-1)[:, None]
    oh1 = (eidx == a1).astype(jnp.float32)
    w1 = jnp.max(probs, axis=-1, keepdims=True)
    probs2 = jnp.where(oh1 > 0, -jnp.inf, probs)
    a2 = jnp.argmax(probs2, axis=-1)[:, None]
    oh2 = (eidx == a2).astype(jnp.float32)
    w2 = jnp.max(probs2, axis=-1, keepdims=True)

    denom = w1 + w2
    combine = (w1 / denom) * oh1 + (w2 / denom) * oh2
    mask = oh1 + oh2

    @pl.when(i == 0)
    def _():
        probs_acc[...] = jnp.zeros_like(probs_acc)
        mask_acc[...] = jnp.zeros_like(mask_acc)

    probs_acc[...] += jnp.sum(probs, axis=0, keepdims=True)
    mask_acc[...] += jnp.sum(mask, axis=0, keepdims=True)

    @pl.when(i == nblocks - 1)
    def _():
        loss_ref[...] = jnp.sum(
            probs_acc[...] * mask_acc[...], keepdims=True) / (_B * _B)

    acc = jnp.dot(combine, be_ref[...],
                  preferred_element_type=jnp.float32)
    for e in range(_E):
        y = jax.lax.dot_general(
            x, we_ref[e],
            dimension_numbers=(((1,), (1,)), ((), ())),
            preferred_element_type=jnp.float32)
        acc = acc + combine[:, e][:, None] * y
    out_ref[...] = acc


def kernel(x, context_vector, Wr, br, We, be, context_weight):
    rb = (br + context_weight * context_vector).reshape(1, _E)
    wrt = Wr.T

    grid = (_B // _BLK,)
    out, loss = pl.pallas_call(
        _moe_kernel,
        grid=grid,
        in_specs=[
            pl.BlockSpec((_BLK, _D), lambda i: (i, 0)),
            pl.BlockSpec((_D, _E), lambda i: (0, 0)),
            pl.BlockSpec((1, _E), lambda i: (0, 0)),
            pl.BlockSpec((_E, _D, _D), lambda i: (0, 0, 0)),
            pl.BlockSpec((_E, _D), lambda i: (0, 0)),
        ],
        out_specs=[
            pl.BlockSpec((_BLK, _D), lambda i: (i, 0)),
            pl.BlockSpec((1, 1), lambda i: (0, 0)),
        ],
        out_shape=[
            jax.ShapeDtypeStruct((_B, _D), jnp.float32),
            jax.ShapeDtypeStruct((1, 1), jnp.float32),
        ],
        scratch_shapes=[
            pltpu.VMEM((1, _E), jnp.float32),
            pltpu.VMEM((1, _E), jnp.float32),
        ],
    )(x, wrt, rb, We, be)
    return out, loss[0, 0]


# speedup vs baseline: 1.0457x; 1.0032x over previous
import jax
import jax.numpy as jnp
from jax.experimental import pallas as pl
from jax.experimental.pallas import tpu as pltpu

_B, _D, _E, _K = 2048, 768, 8, 2
_BLK = 2048


def _moe_kernel(x_ref, wrt_ref, rb_ref, we_ref, be_ref,
                out_ref, loss_ref, probs_acc, mask_acc):
    i = pl.program_id(0)
    nblocks = pl.num_programs(0)

    x = x_ref[...]

    logits = jnp.dot(x, wrt_ref[...],
                     preferred_element_type=jnp.float32) + rb_ref[...]
    m = jnp.max(logits, axis=-1, keepdims=True)
    ex = jnp.exp(logits - m)
    probs = ex / jnp.sum(ex, axis=-1, keepdims=True)

    eidx = jax.lax.broadcasted_iota(jnp.int32, probs.shape, 1)
    a1 = jnp.argmax(probs, axis=-1)[:, None]
    oh1 = (eidx == a1).astype(jnp.float32)
    w1 = jnp.max(probs, axis=-1, keepdims=True)
    probs2 = jnp.where(oh1 > 0, -jnp.inf, probs)
    a2 = jnp.argmax(probs2, axis=-1)[:, None]
    oh2 = (eidx == a2).astype(jnp.float32)
    w2 = jnp.max(probs2, axis=-1, keepdims=True)

    denom = w1 + w2
    combine = (w1 / denom) * oh1 + (w2 / denom) * oh2
    mask = oh1 + oh2

    @pl.when(i == 0)
    def _():
        probs_acc[...] = jnp.zeros_like(probs_acc)
        mask_acc[...] = jnp.zeros_like(mask_acc)

    probs_acc[...] += jnp.sum(probs, axis=0, keepdims=True)
    mask_acc[...] += jnp.sum(mask, axis=0, keepdims=True)

    @pl.when(i == nblocks - 1)
    def _():
        loss_ref[...] = jnp.sum(
            probs_acc[...] * mask_acc[...], keepdims=True) / (_B * _B)

    acc = jnp.dot(combine, be_ref[...],
                  preferred_element_type=jnp.float32)
    xb = x.astype(jnp.bfloat16)
    for e in range(_E):
        y = jax.lax.dot_general(
            xb, we_ref[e].astype(jnp.bfloat16),
            dimension_numbers=(((1,), (1,)), ((), ())),
            preferred_element_type=jnp.float32)
        acc = acc + combine[:, e][:, None] * y
    out_ref[...] = acc


def kernel(x, context_vector, Wr, br, We, be, context_weight):
    rb = (br + context_weight * context_vector).reshape(1, _E)
    wrt = Wr.T

    grid = (_B // _BLK,)
    out, loss = pl.pallas_call(
        _moe_kernel,
        grid=grid,
        in_specs=[
            pl.BlockSpec((_BLK, _D), lambda i: (i, 0)),
            pl.BlockSpec((_D, _E), lambda i: (0, 0)),
            pl.BlockSpec((1, _E), lambda i: (0, 0)),
            pl.BlockSpec((_E, _D, _D), lambda i: (0, 0, 0)),
            pl.BlockSpec((_E, _D), lambda i: (0, 0)),
        ],
        out_specs=[
            pl.BlockSpec((_BLK, _D), lambda i: (i, 0)),
            pl.BlockSpec((1, 1), lambda i: (0, 0)),
        ],
        out_shape=[
            jax.ShapeDtypeStruct((_B, _D), jnp.float32),
            jax.ShapeDtypeStruct((1, 1), jnp.float32),
        ],
        scratch_shapes=[
            pltpu.VMEM((1, _E), jnp.float32),
            pltpu.VMEM((1, _E), jnp.float32),
        ],
    )(x, wrt, rb, We, be)
    return out, loss[0, 0]
